# baseline (speedup 1.0000x reference)
import jax
import jax.numpy as jnp
from jax import lax
from jax.experimental import pallas as pl
from jax.experimental.pallas import tpu as pltpu
from jax.experimental.pallas import tpu_sc as plsc

N = 10000
E = 320000
IN_F = 128
HD = 128
H = 8
DH = 16
RW = HD + DH
NC = 2
NS = 16
NW = NC * NS
EPW = E // NW
C = 80
NCHUNK = EPW // C
RPT = 624
TAIL = N - NS * RPT
ZR = 48


def _tc1_body(vert_ref, w_ref, asrc_ref, adst_ref, g2_ref, q_ref):
    v = vert_ref[...]
    g = jnp.dot(v, w_ref[...], preferred_element_type=jnp.float32)
    hrow = lax.broadcasted_iota(jnp.int32, (HD, H), 0) // DH
    hcol = lax.broadcasted_iota(jnp.int32, (HD, H), 1)
    mask = (hrow == hcol).astype(jnp.float32)
    ssrc = jnp.dot(g * asrc_ref[...], mask,
                   preferred_element_type=jnp.float32)
    sdst = jnp.dot(g * adst_ref[...], mask,
                   preferred_element_type=jnp.float32)
    zero8 = jnp.zeros_like(ssrc)
    g2_ref[...] = jnp.concatenate([g, ssrc, zero8], axis=1)
    q_ref[...] = jnp.concatenate([sdst, zero8], axis=1)


def _tc1(vert, w2, a_src, a_dst):
    return pl.pallas_call(
        _tc1_body,
        out_shape=[
            jax.ShapeDtypeStruct((N, RW), jnp.float32),
            jax.ShapeDtypeStruct((N, DH), jnp.float32),
        ],
    )(vert, w2, a_src, a_dst)


def _sc_body(src_hbm, dst_hbm, g2_hbm, q_hbm, u2_out,
             shared_u2, src_idx, dst_idx, q_buf, row_buf, zbuf,
             sem_g, sem_q, sem_s):
    c = lax.axis_index("c")
    s = lax.axis_index("s")
    wid = c * NS + s

    def _z(i, carry):
        zbuf[i // (RW // DH), pl.ds((i % (RW // DH)) * DH, DH)] = (
            jnp.zeros((DH,), jnp.float32))
        return carry
    lax.fori_loop(0, ZR * (RW // DH), _z, 0)

    def _zspm(j, carry):
        pltpu.sync_copy(zbuf, shared_u2.at[pl.ds(s * RPT + j * ZR, ZR)])
        return carry
    lax.fori_loop(0, RPT // ZR, _zspm, 0)

    @pl.when(s == 0)
    def _ztail():
        pltpu.sync_copy(zbuf.at[pl.ds(0, TAIL)],
                        shared_u2.at[pl.ds(NS * RPT, TAIL)])
    plsc.subcore_barrier()

    ebase = wid * EPW

    def _issue_gathers(slot, j):
        base = ebase + j * C
        pltpu.sync_copy(src_hbm.at[pl.ds(base, C)], src_idx.at[slot])
        pltpu.sync_copy(dst_hbm.at[pl.ds(base, C)], dst_idx.at[slot])
        pltpu.async_copy(g2_hbm.at[src_idx.at[slot]], row_buf.at[slot],
                         sem_g)
        pltpu.async_copy(q_hbm.at[dst_idx.at[slot]], q_buf.at[slot], sem_q)

    def _wait_gathers(slot):
        pltpu.make_async_copy(g2_hbm.at[src_idx.at[slot]],
                              row_buf.at[slot], sem_g).wait()
        pltpu.make_async_copy(q_hbm.at[dst_idx.at[slot]],
                              q_buf.at[slot], sem_q).wait()

    def _wait_scatter(slot):
        pltpu.make_async_copy(row_buf.at[slot],
                              shared_u2.at[dst_idx.at[slot]], sem_s).wait()

    _issue_gathers(0, 0)

    def _chunk(j, carry):
        cur = lax.rem(j, 2)
        nxt = 1 - cur

        @pl.when(j >= 1)
        def _():
            _wait_scatter(nxt)

        @pl.when(j + 1 < NCHUNK)
        def _():
            _issue_gathers(nxt, j + 1)

        _wait_gathers(cur)

        @plsc.parallel_loop(0, C, step=1, unroll=8)
        def _edge(e):
            t = row_buf[cur, e, pl.ds(HD, DH)] + q_buf[cur, e, :]
            t = jnp.where(t > 0.0, t, 0.2 * t)
            w = jnp.exp(t)
            for h in range(H):
                blk = row_buf[cur, e, pl.ds(h * DH, DH)]
                row_buf[cur, e, pl.ds(h * DH, DH)] = blk * w[h]
            row_buf[cur, e, pl.ds(HD, DH)] = w

        pltpu.async_copy(row_buf.at[cur], shared_u2.at[dst_idx.at[cur]],
                         sem_s, add=True)
        return carry
    lax.fori_loop(0, NCHUNK, _chunk, 0)
    _wait_scatter((NCHUNK - 1) % 2)
    plsc.subcore_barrier()

    pltpu.sync_copy(shared_u2.at[pl.ds(s * RPT, RPT)],
                    u2_out.at[c, pl.ds(s * RPT, RPT)])

    @pl.when(s == 0)
    def _dtail():
        pltpu.sync_copy(shared_u2.at[pl.ds(NS * RPT, TAIL)],
                        u2_out.at[c, pl.ds(NS * RPT, TAIL)])


def _sc(src, dst, g2, q):
    mesh = plsc.VectorSubcoreMesh(core_axis_name="c", subcore_axis_name="s")
    fn = pl.kernel(
        _sc_body,
        out_type=jax.ShapeDtypeStruct((NC, N, RW), jnp.float32),
        mesh=mesh,
        compiler_params=pltpu.CompilerParams(use_tc_tiling_on_sc=False),
        scratch_types=[
            pltpu.VMEM_SHARED((N, RW), jnp.float32),
            pltpu.VMEM((2, C), jnp.int32),
            pltpu.VMEM((2, C), jnp.int32),
            pltpu.VMEM((2, C, DH), jnp.float32),
            pltpu.VMEM((2, C, RW), jnp.float32),
            pltpu.VMEM((ZR, RW), jnp.float32),
            pltpu.SemaphoreType.DMA,
            pltpu.SemaphoreType.DMA,
            pltpu.SemaphoreType.DMA,
        ],
    )
    return fn(src, dst, g2, q)


def _tc2_body(u2_ref, o_ref):
    u = u2_ref[0, :, :HD] + u2_ref[1, :, :HD]
    d = u2_ref[0, :, HD:HD + H] + u2_ref[1, :, HD:HD + H]
    brow = lax.broadcasted_iota(jnp.int32, (H, HD), 0)
    bcol = lax.broadcasted_iota(jnp.int32, (H, HD), 1) // DH
    bmat = (brow == bcol).astype(jnp.float32)
    dfull = jnp.dot(d, bmat, preferred_element_type=jnp.float32)
    x = u / (dfull + 1e-16)
    o_ref[...] = jnp.where(x > 0.0, x, jnp.exp(jnp.minimum(x, 0.0)) - 1.0)


def _tc2(u2_part):
    return pl.pallas_call(
        _tc2_body,
        out_shape=jax.ShapeDtypeStruct((N, HD), jnp.float32),
    )(u2_part)


def kernel(vert, edge, W, a_src, a_dst):
    w2 = W.reshape(IN_F, HD)
    src = edge[0]
    dst = edge[1]
    g2, q = _tc1(vert, w2, a_src.reshape(1, HD), a_dst.reshape(1, HD))
    u2_part = _sc(src, dst, g2, q)
    return _tc2(u2_part)


# speedup vs baseline: 1.4720x; 1.4720x over previous
import jax
import jax.numpy as jnp
from jax import lax
from jax.experimental import pallas as pl
from jax.experimental.pallas import tpu as pltpu
from jax.experimental.pallas import tpu_sc as plsc

N = 10000
E = 320000
IN_F = 128
HD = 128
H = 8
DH = 16
RW = HD + DH
NC = 2
NS = 16
NW = NC * NS
EPW = E // NW
C = 80
NCHUNK = EPW // C
RPT = 624
TAIL = N - NS * RPT
ZR = 16
NSLOT = 3


def _tc1_body(vert_ref, w_ref, asrc_ref, adst_ref, g2_ref, q_ref):
    v = vert_ref[...]
    g = jnp.dot(v, w_ref[...], preferred_element_type=jnp.float32)
    hrow = lax.broadcasted_iota(jnp.int32, (HD, H), 0) // DH
    hcol = lax.broadcasted_iota(jnp.int32, (HD, H), 1)
    mask = (hrow == hcol).astype(jnp.float32)
    ssrc = jnp.dot(g * asrc_ref[...], mask,
                   preferred_element_type=jnp.float32)
    sdst = jnp.dot(g * adst_ref[...], mask,
                   preferred_element_type=jnp.float32)
    zero8 = jnp.zeros_like(ssrc)
    g2_ref[...] = jnp.concatenate([g, ssrc, zero8], axis=1)
    q_ref[...] = jnp.concatenate([sdst, zero8], axis=1)


def _tc1(vert, w2, a_src, a_dst):
    return pl.pallas_call(
        _tc1_body,
        out_shape=[
            jax.ShapeDtypeStruct((N, RW), jnp.float32),
            jax.ShapeDtypeStruct((N, DH), jnp.float32),
        ],
    )(vert, w2, a_src, a_dst)


def _sc_body(src_hbm, dst_hbm, g2_hbm, q_hbm, u2_out,
             shared_u2, src_idx, dst_idx, q_buf, row_buf,
             sem_g, sem_q, sem_s):
    c = lax.axis_index("c")
    s = lax.axis_index("s")
    wid = c * NS + s

    def _z(i, carry):
        row_buf[0, i // (RW // DH), pl.ds((i % (RW // DH)) * DH, DH)] = (
            jnp.zeros((DH,), jnp.float32))
        return carry
    lax.fori_loop(0, ZR * (RW // DH), _z, 0)

    def _zspm(j, carry):
        pltpu.sync_copy(row_buf.at[0, pl.ds(0, ZR)],
                        shared_u2.at[pl.ds(s * RPT + j * ZR, ZR)])
        return carry
    lax.fori_loop(0, RPT // ZR, _zspm, 0)

    @pl.when(s == 0)
    def _ztail():
        pltpu.sync_copy(row_buf.at[0, pl.ds(0, TAIL)],
                        shared_u2.at[pl.ds(NS * RPT, TAIL)])
    plsc.subcore_barrier()

    ebase = wid * EPW

    def _issue_gathers(slot, j):
        base = ebase + j * C
        pltpu.sync_copy(src_hbm.at[pl.ds(base, C)], src_idx.at[slot])
        pltpu.sync_copy(dst_hbm.at[pl.ds(base, C)], dst_idx.at[slot])
        pltpu.async_copy(g2_hbm.at[src_idx.at[slot]], row_buf.at[slot],
                         sem_g)
        pltpu.async_copy(q_hbm.at[dst_idx.at[slot]], q_buf.at[slot], sem_q)

    def _wait_gathers(slot):
        pltpu.make_async_copy(g2_hbm.at[src_idx.at[slot]],
                              row_buf.at[slot], sem_g).wait()
        pltpu.make_async_copy(q_hbm.at[dst_idx.at[slot]],
                              q_buf.at[slot], sem_q).wait()

    def _wait_scatter(slot):
        pltpu.make_async_copy(row_buf.at[slot],
                              shared_u2.at[dst_idx.at[slot]], sem_s).wait()

    _issue_gathers(0, 0)

    def _chunk(j, carry):
        cur = lax.rem(j, NSLOT)
        pre = lax.rem(j + 1, NSLOT)

        @pl.when(j >= 2)
        def _():
            _wait_scatter(pre)

        @pl.when(j + 1 < NCHUNK)
        def _():
            _issue_gathers(pre, j + 1)

        _wait_gathers(cur)

        @plsc.parallel_loop(0, C, step=1, unroll=4)
        def _edge(e):
            t = row_buf[cur, e, pl.ds(HD, DH)] + q_buf[cur, e, :]
            t = jnp.where(t > 0.0, t, 0.2 * t)
            w = jnp.exp(t)
            for h in range(H):
                blk = row_buf[cur, e, pl.ds(h * DH, DH)]
                row_buf[cur, e, pl.ds(h * DH, DH)] = blk * w[h]
            row_buf[cur, e, pl.ds(HD, DH)] = w

        pltpu.async_copy(row_buf.at[cur], shared_u2.at[dst_idx.at[cur]],
                         sem_s, add=True)
        return carry
    lax.fori_loop(0, NCHUNK, _chunk, 0)
    _wait_scatter((NCHUNK - 2) % NSLOT)
    _wait_scatter((NCHUNK - 1) % NSLOT)
    plsc.subcore_barrier()

    pltpu.sync_copy(shared_u2.at[pl.ds(s * RPT, RPT)],
                    u2_out.at[c, pl.ds(s * RPT, RPT)])

    @pl.when(s == 0)
    def _dtail():
        pltpu.sync_copy(shared_u2.at[pl.ds(NS * RPT, TAIL)],
                        u2_out.at[c, pl.ds(NS * RPT, TAIL)])


def _sc(src, dst, g2, q):
    mesh = plsc.VectorSubcoreMesh(core_axis_name="c", subcore_axis_name="s")
    fn = pl.kernel(
        _sc_body,
        out_type=jax.ShapeDtypeStruct((NC, N, RW), jnp.float32),
        mesh=mesh,
        compiler_params=pltpu.CompilerParams(use_tc_tiling_on_sc=False),
        scratch_types=[
            pltpu.VMEM_SHARED((N, RW), jnp.float32),
            pltpu.VMEM((NSLOT, C), jnp.int32),
            pltpu.VMEM((NSLOT, C), jnp.int32),
            pltpu.VMEM((NSLOT, C, DH), jnp.float32),
            pltpu.VMEM((NSLOT, C, RW), jnp.float32),
            pltpu.SemaphoreType.DMA,
            pltpu.SemaphoreType.DMA,
            pltpu.SemaphoreType.DMA,
        ],
    )
    return fn(src, dst, g2, q)


def _tc2_body(u2_ref, o_ref):
    u = u2_ref[0, :, :HD] + u2_ref[1, :, :HD]
    d = u2_ref[0, :, HD:HD + H] + u2_ref[1, :, HD:HD + H]
    brow = lax.broadcasted_iota(jnp.int32, (H, HD), 0)
    bcol = lax.broadcasted_iota(jnp.int32, (H, HD), 1) // DH
    bmat = (brow == bcol).astype(jnp.float32)
    dfull = jnp.dot(d, bmat, preferred_element_type=jnp.float32)
    x = u / (dfull + 1e-16)
    o_ref[...] = jnp.where(x > 0.0, x, jnp.exp(jnp.minimum(x, 0.0)) - 1.0)


def _tc2(u2_part):
    return pl.pallas_call(
        _tc2_body,
        out_shape=jax.ShapeDtypeStruct((N, HD), jnp.float32),
    )(u2_part)


def kernel(vert, edge, W, a_src, a_dst):
    w2 = W.reshape(IN_F, HD)
    src = edge[0]
    dst = edge[1]
    g2, q = _tc1(vert, w2, a_src.reshape(1, HD), a_dst.reshape(1, HD))
    u2_part = _sc(src, dst, g2, q)
    return _tc2(u2_part)


# speedup vs baseline: 1.7407x; 1.1825x over previous
import jax
import jax.numpy as jnp
from jax import lax
from jax.experimental import pallas as pl
from jax.experimental.pallas import tpu as pltpu
from jax.experimental.pallas import tpu_sc as plsc

N = 10000
E = 320000
IN_F = 128
HD = 128
H = 8
DH = 16
RW = HD + DH
NC = 2
NS = 16
NW = NC * NS
EPW = E // NW
C = 80
NCHUNK = EPW // C
RPT = 624
TAIL = N - NS * RPT
ZR = 16
NSLOT = 3
NIDX = 5


def _tc1_body(vert_ref, w_ref, asrc_ref, adst_ref, g2_ref, q_ref):
    v = vert_ref[...]
    g = jnp.dot(v, w_ref[...], preferred_element_type=jnp.float32)
    hrow = lax.broadcasted_iota(jnp.int32, (HD, H), 0) // DH
    hcol = lax.broadcasted_iota(jnp.int32, (HD, H), 1)
    mask = (hrow == hcol).astype(jnp.float32)
    ssrc = jnp.dot(g * asrc_ref[...], mask,
                   preferred_element_type=jnp.float32)
    sdst = jnp.dot(g * adst_ref[...], mask,
                   preferred_element_type=jnp.float32)
    zero8 = jnp.zeros_like(ssrc)
    g2_ref[...] = jnp.concatenate([g, ssrc, zero8], axis=1)
    q_ref[...] = jnp.concatenate([sdst, zero8], axis=1)


def _tc1(vert, w2, a_src, a_dst):
    return pl.pallas_call(
        _tc1_body,
        out_shape=[
            jax.ShapeDtypeStruct((N, RW), jnp.float32),
            jax.ShapeDtypeStruct((N, DH), jnp.float32),
        ],
    )(vert, w2, a_src, a_dst)


def _sc_body(src_hbm, dst_hbm, g2_hbm, q_hbm, u2_out,
             shared_u2, src_idx, dst_idx, q_buf, row_buf,
             sem_g, sem_q, sem_s, sem_i):
    c = lax.axis_index("c")
    s = lax.axis_index("s")
    wid = c * NS + s

    def _z(i, carry):
        row_buf[0, i // (RW // DH), pl.ds((i % (RW // DH)) * DH, DH)] = (
            jnp.zeros((DH,), jnp.float32))
        return carry
    lax.fori_loop(0, ZR * (RW // DH), _z, 0)

    def _zspm(j, carry):
        pltpu.sync_copy(row_buf.at[0, pl.ds(0, ZR)],
                        shared_u2.at[pl.ds(s * RPT + j * ZR, ZR)])
        return carry
    lax.fori_loop(0, RPT // ZR, _zspm, 0)

    @pl.when(s == 0)
    def _ztail():
        pltpu.sync_copy(row_buf.at[0, pl.ds(0, TAIL)],
                        shared_u2.at[pl.ds(NS * RPT, TAIL)])
    plsc.subcore_barrier()

    ebase = wid * EPW

    def _issue_idx(islot, j):
        base = ebase + j * C
        pltpu.async_copy(src_hbm.at[pl.ds(base, C)], src_idx.at[islot],
                         sem_i)
        pltpu.async_copy(dst_hbm.at[pl.ds(base, C)], dst_idx.at[islot],
                         sem_i)

    def _wait_idx(islot, j):
        base = ebase + j * C
        pltpu.make_async_copy(src_hbm.at[pl.ds(base, C)],
                              src_idx.at[islot], sem_i).wait()
        pltpu.make_async_copy(dst_hbm.at[pl.ds(base, C)],
                              dst_idx.at[islot], sem_i).wait()

    def _issue_gathers(slot, islot):
        pltpu.async_copy(g2_hbm.at[src_idx.at[islot]], row_buf.at[slot],
                         sem_g)
        pltpu.async_copy(q_hbm.at[dst_idx.at[islot]], q_buf.at[slot],
                         sem_q)

    def _wait_gathers(slot, islot):
        pltpu.make_async_copy(g2_hbm.at[src_idx.at[islot]],
                              row_buf.at[slot], sem_g).wait()
        pltpu.make_async_copy(q_hbm.at[dst_idx.at[islot]],
                              q_buf.at[slot], sem_q).wait()

    def _wait_scatter(slot, islot):
        pltpu.make_async_copy(row_buf.at[slot],
                              shared_u2.at[dst_idx.at[islot]], sem_s).wait()

    _issue_idx(0, 0)
    _wait_idx(0, 0)
    _issue_idx(1, 1)
    _issue_gathers(0, 0)

    def _chunk(j, carry):
        cur = lax.rem(j, NSLOT)
        pre = lax.rem(j + 1, NSLOT)
        icur = lax.rem(j, NIDX)
        inxt = lax.rem(j + 1, NIDX)
        ipre = lax.rem(j + 2, NIDX)

        @pl.when(j + 2 < NCHUNK)
        def _():
            _issue_idx(ipre, j + 2)

        @pl.when(j >= 2)
        def _():
            _wait_scatter(pre, lax.rem(j - 2, NIDX))

        @pl.when(j + 1 < NCHUNK)
        def _():
            _wait_idx(inxt, j + 1)
            _issue_gathers(pre, inxt)

        _wait_gathers(cur, icur)

        @plsc.parallel_loop(0, C, step=1, unroll=4)
        def _edge(e):
            t = row_buf[cur, e, pl.ds(HD, DH)] + q_buf[cur, e, :]
            t = jnp.where(t > 0.0, t, 0.2 * t)
            w = jnp.exp(t)
            for h in range(H):
                blk = row_buf[cur, e, pl.ds(h * DH, DH)]
                row_buf[cur, e, pl.ds(h * DH, DH)] = blk * w[h]
            row_buf[cur, e, pl.ds(HD, DH)] = w

        pltpu.async_copy(row_buf.at[cur], shared_u2.at[dst_idx.at[icur]],
                         sem_s, add=True)
        return carry
    lax.fori_loop(0, NCHUNK, _chunk, 0)
    _wait_scatter((NCHUNK - 2) % NSLOT, (NCHUNK - 2) % NIDX)
    _wait_scatter((NCHUNK - 1) % NSLOT, (NCHUNK - 1) % NIDX)
    plsc.subcore_barrier()

    pltpu.sync_copy(shared_u2.at[pl.ds(s * RPT, RPT)],
                    u2_out.at[c, pl.ds(s * RPT, RPT)])

    @pl.when(s == 0)
    def _dtail():
        pltpu.sync_copy(shared_u2.at[pl.ds(NS * RPT, TAIL)],
                        u2_out.at[c, pl.ds(NS * RPT, TAIL)])


def _sc(src, dst, g2, q):
    mesh = plsc.VectorSubcoreMesh(core_axis_name="c", subcore_axis_name="s")
    fn = pl.kernel(
        _sc_body,
        out_type=jax.ShapeDtypeStruct((NC, N, RW), jnp.float32),
        mesh=mesh,
        compiler_params=pltpu.CompilerParams(use_tc_tiling_on_sc=False),
        scratch_types=[
            pltpu.VMEM_SHARED((N, RW), jnp.float32),
            pltpu.VMEM((NIDX, C), jnp.int32),
            pltpu.VMEM((NIDX, C), jnp.int32),
            pltpu.VMEM((NSLOT, C, DH), jnp.float32),
            pltpu.VMEM((NSLOT, C, RW), jnp.float32),
            pltpu.SemaphoreType.DMA,
            pltpu.SemaphoreType.DMA,
            pltpu.SemaphoreType.DMA,
            pltpu.SemaphoreType.DMA,
        ],
    )
    return fn(src, dst, g2, q)


def _tc2_body(u2_ref, o_ref):
    u = u2_ref[0, :, :HD] + u2_ref[1, :, :HD]
    d = u2_ref[0, :, HD:HD + H] + u2_ref[1, :, HD:HD + H]
    brow = lax.broadcasted_iota(jnp.int32, (H, HD), 0)
    bcol = lax.broadcasted_iota(jnp.int32, (H, HD), 1) // DH
    bmat = (brow == bcol).astype(jnp.float32)
    dfull = jnp.dot(d, bmat, preferred_element_type=jnp.float32)
    x = u / (dfull + 1e-16)
    o_ref[...] = jnp.where(x > 0.0, x, jnp.exp(jnp.minimum(x, 0.0)) - 1.0)


def _tc2(u2_part):
    return pl.pallas_call(
        _tc2_body,
        out_shape=jax.ShapeDtypeStruct((N, HD), jnp.float32),
    )(u2_part)


def kernel(vert, edge, W, a_src, a_dst):
    w2 = W.reshape(IN_F, HD)
    src = edge[0]
    dst = edge[1]
    g2, q = _tc1(vert, w2, a_src.reshape(1, HD), a_dst.reshape(1, HD))
    u2_part = _sc(src, dst, g2, q)
    return _tc2(u2_part)
